# baseline (speedup 1.0000x reference)
import functools

import jax
import jax.numpy as jnp
from jax import lax
from jax.experimental import pallas as pl
from jax.experimental.pallas import tpu as pltpu
from jax.experimental.pallas import tpu_sc as plsc

B = 1024
D = 256
NT = D * (D + 1) // 2
NW = 32
NB = B // NW
NB2 = NB // 2

_OFF = [0] * (D + 1)
for _r in range(D):
    _OFF[_r + 1] = _OFF[_r] + D - _r


def _tec_body(x_hbm, out_hbm, in_a, in_b, out0, out1, sem_a, sem_b,
              sem_o0, sem_o1):
    wid = lax.axis_index("s") * 2 + lax.axis_index("c")

    def src_a(b):
        return x_hbm.at[b, pl.ds(0, 128)]

    def src_b(b):
        return x_hbm.at[b, pl.ds(128, 128), pl.ds(128, 128)]

    def row_copy(in_v, r, c0, out_v, dst):
        for k in range(8):
            out_v[pl.ds(dst + 16 * k, 16)] = in_v[r, pl.ds(c0 + 16 * k, 16)]

    def do_batch(i, b, b_next, out_v, sem_o, have_next):
        pltpu.make_async_copy(src_a(b), in_a, sem_a).wait()

        @pl.when(i > 0)
        def _():
            pltpu.make_async_copy(out_v.at[pl.ds(0, NT)], out_hbm.at[b],
                                  sem_o).wait()

        def loop1(j, dst):
            row_copy(in_a, 127 - j, 0, out_v, dst)
            return dst - 129 - j

        lax.fori_loop(0, 128, loop1, _OFF[127] - 127)

        def loop2(r, dst):
            row_copy(in_a, r, 128, out_v, dst)
            return dst + 255 - r

        lax.fori_loop(0, 128, loop2, 128)

        @pl.when(have_next)
        def _():
            pltpu.async_copy(src_a(b_next), in_a, sem_a)

        pltpu.make_async_copy(src_b(b), in_b, sem_b).wait()

        def loop3(j, dst):
            row_copy(in_b, 127 - j, 0, out_v, dst)
            return dst - 1 - j

        lax.fori_loop(0, 128, loop3, _OFF[255] - 127)

        @pl.when(have_next)
        def _():
            pltpu.async_copy(src_b(b_next), in_b, sem_b)

        pltpu.async_copy(out_v.at[pl.ds(0, NT)], out_hbm.at[b], sem_o)

    pltpu.async_copy(src_a(wid), in_a, sem_a)
    pltpu.async_copy(src_b(wid), in_b, sem_b)

    def pair_loop(i, carry):
        b0 = (2 * i) * NW + wid
        b1 = (2 * i + 1) * NW + wid
        b2 = (2 * i + 2) * NW + wid
        do_batch(i, b0, b1, out0, sem_o0, True)
        do_batch(i, b1, b2, out1, sem_o1, i + 1 < NB2)
        return carry

    lax.fori_loop(0, NB2, pair_loop, 0)

    last0 = (NB - 2) * NW + wid
    last1 = (NB - 1) * NW + wid
    pltpu.make_async_copy(out0.at[pl.ds(0, NT)], out_hbm.at[last0],
                          sem_o0).wait()
    pltpu.make_async_copy(out1.at[pl.ds(0, NT)], out_hbm.at[last1],
                          sem_o1).wait()


@jax.jit
def kernel(x):
    mesh = plsc.VectorSubcoreMesh(core_axis_name="c", subcore_axis_name="s")
    run = functools.partial(
        pl.kernel,
        mesh=mesh,
        out_type=jax.ShapeDtypeStruct((B, NT), jnp.float32),
        scratch_types=[
            pltpu.VMEM((128, D), jnp.float32),
            pltpu.VMEM((128, 128), jnp.float32),
            pltpu.VMEM((NT,), jnp.float32),
            pltpu.VMEM((NT,), jnp.float32),
            pltpu.SemaphoreType.DMA,
            pltpu.SemaphoreType.DMA,
            pltpu.SemaphoreType.DMA,
            pltpu.SemaphoreType.DMA,
        ],
        compiler_params=pltpu.CompilerParams(use_tc_tiling_on_sc=True),
    )(_tec_body)
    return run(x)


# speedup vs baseline: 1.2243x; 1.2243x over previous
import functools

import jax
import jax.numpy as jnp
from jax import lax
from jax.experimental import pallas as pl
from jax.experimental.pallas import tpu as pltpu
from jax.experimental.pallas import tpu_sc as plsc

B = 1024
D = 256
NT = D * (D + 1) // 2
NW = 32
NB = B // NW
NB2 = NB // 2

_OFF = [0] * (D + 1)
for _r in range(D):
    _OFF[_r + 1] = _OFF[_r] + D - _r


def _ilv(lst, k=4):
    return [x for j in range(k) for x in lst[j::k]]


def _chunks_half_a():
    cluster, main = [], []
    for r in range(128):
        dst = _OFF[r]
        plen = 128 - r
        nf, rem = plen // 16, plen % 16
        if plen < 16:
            cluster.append((r, 128 - 16, dst + plen - 16))
        else:
            for k in range(nf):
                main.append((r, r + 16 * k, dst + 16 * k))
            if rem:
                main.append((r, 128 - 16, dst + plen - 16))
    return cluster[::-1] + _ilv(main)


def _chunks_half_b():
    cluster, main = [], []
    for r in range(128, 256):
        rl = r - 128
        plen = 128 - rl
        nf, rem = plen // 16, plen % 16
        dst = _OFF[r]
        if plen < 16:
            cluster.append((rl, 128 - 16, dst + plen - 16))
        else:
            for k in range(nf):
                main.append((rl, rl + 16 * k, dst + 16 * k))
            if rem:
                main.append((rl, 128 - 16, dst + plen - 16))
    return cluster[::-1] + _ilv(main)


_CHUNKS_A = _chunks_half_a()
_CHUNKS_B = _chunks_half_b()


def _tec_body(x_hbm, out_hbm, in_a, in_b, out0, out1, sem_a, sem_b,
              sem_o0, sem_o1):
    wid = lax.axis_index("s") * 2 + lax.axis_index("c")

    def src_a(b):
        return x_hbm.at[b, pl.ds(0, 128)]

    def src_b(b):
        return x_hbm.at[b, pl.ds(128, 128), pl.ds(128, 128)]

    def emit(chunks, in_v, out_v):
        for r, c, d in chunks:
            out_v[pl.ds(d, 16)] = in_v[r, pl.ds(c, 16)]

    def do_batch(i, b, b_next, out_v, sem_o, have_next):
        pltpu.make_async_copy(src_a(b), in_a, sem_a).wait()

        @pl.when(i > 0)
        def _():
            pltpu.make_async_copy(out_v.at[pl.ds(0, NT)], out_hbm.at[b],
                                  sem_o).wait()

        emit(_CHUNKS_A, in_a, out_v)

        def p2_loop(r, dst):
            for k in range(8):
                out_v[pl.ds(dst + 16 * k, 16)] = in_a[r, pl.ds(128 + 16 * k, 16)]
            return dst + 255 - r

        lax.fori_loop(0, 128, p2_loop, 128)

        @pl.when(have_next)
        def _():
            pltpu.async_copy(src_a(b_next), in_a, sem_a)

        pltpu.make_async_copy(src_b(b), in_b, sem_b).wait()
        emit(_CHUNKS_B, in_b, out_v)

        @pl.when(have_next)
        def _():
            pltpu.async_copy(src_b(b_next), in_b, sem_b)

        pltpu.async_copy(out_v.at[pl.ds(0, NT)], out_hbm.at[b], sem_o)

    pltpu.async_copy(src_a(wid), in_a, sem_a)
    pltpu.async_copy(src_b(wid), in_b, sem_b)

    def pair_loop(i, carry):
        b0 = (2 * i) * NW + wid
        b1 = (2 * i + 1) * NW + wid
        b2 = (2 * i + 2) * NW + wid
        do_batch(i, b0, b1, out0, sem_o0, True)
        do_batch(i, b1, b2, out1, sem_o1, i + 1 < NB2)
        return carry

    lax.fori_loop(0, NB2, pair_loop, 0)

    last0 = (NB - 2) * NW + wid
    last1 = (NB - 1) * NW + wid
    pltpu.make_async_copy(out0.at[pl.ds(0, NT)], out_hbm.at[last0],
                          sem_o0).wait()
    pltpu.make_async_copy(out1.at[pl.ds(0, NT)], out_hbm.at[last1],
                          sem_o1).wait()


@jax.jit
def kernel(x):
    mesh = plsc.VectorSubcoreMesh(core_axis_name="c", subcore_axis_name="s")
    run = functools.partial(
        pl.kernel,
        mesh=mesh,
        out_type=jax.ShapeDtypeStruct((B, NT), jnp.float32),
        scratch_types=[
            pltpu.VMEM((128, D), jnp.float32),
            pltpu.VMEM((128, 128), jnp.float32),
            pltpu.VMEM((NT + 16,), jnp.float32),
            pltpu.VMEM((NT + 16,), jnp.float32),
            pltpu.SemaphoreType.DMA,
            pltpu.SemaphoreType.DMA,
            pltpu.SemaphoreType.DMA,
            pltpu.SemaphoreType.DMA,
        ],
        compiler_params=pltpu.CompilerParams(use_tc_tiling_on_sc=True),
    )(_tec_body)
    return run(x)
